# baseline (speedup 1.0000x reference)
import jax
import jax.numpy as jnp
from jax.experimental import pallas as pl
from jax.experimental.pallas import tpu as pltpu


_COL_START = 26
_COL_COUNT = 100
_N_ROWS = 16384
_NBLK = 8
_BLOCK = _N_ROWS // _NBLK


def _body(in_hbm, out_hbm, ibuf, obuf, isem, osem):
    def in_copy(i):
        return pltpu.make_async_copy(
            in_hbm.at[pl.ds(i * _BLOCK, _BLOCK), :],
            ibuf.at[i],
            isem.at[i],
        )

    def out_copy(i):
        return pltpu.make_async_copy(
            obuf.at[i],
            out_hbm.at[pl.ds(i * _BLOCK, _BLOCK), :],
            osem.at[i],
        )

    in_copy(0).start()
    in_copy(1).start()
    for i in range(_NBLK):
        in_copy(i).wait()
        obuf[i] = ibuf[i][:, _COL_START:_COL_START + _COL_COUNT]
        out_copy(i).start()
        if i + 2 < _NBLK:
            in_copy(i + 2).start()
    for i in range(_NBLK):
        out_copy(i).wait()


def kernel(inputs):
    n_rows, n_cols = inputs.shape
    return pl.pallas_call(
        _body,
        in_specs=[pl.BlockSpec(memory_space=pltpu.MemorySpace.HBM)],
        out_specs=pl.BlockSpec(memory_space=pltpu.MemorySpace.HBM),
        out_shape=jax.ShapeDtypeStruct((n_rows, _COL_COUNT), jnp.float32),
        scratch_shapes=[
            pltpu.VMEM((_NBLK, _BLOCK, 126), jnp.float32),
            pltpu.VMEM((_NBLK, _BLOCK, _COL_COUNT), jnp.float32),
            pltpu.SemaphoreType.DMA((_NBLK,)),
            pltpu.SemaphoreType.DMA((_NBLK,)),
        ],
    )(inputs)


# speedup vs baseline: 1.2121x; 1.2121x over previous
import jax
import jax.numpy as jnp
from jax.experimental import pallas as pl
from jax.experimental.pallas import tpu as pltpu


_COL_START = 26
_COL_COUNT = 100
_N_ROWS = 16384
_NBLK = 8
_BLOCK = _N_ROWS // _NBLK


def _body(in_hbm, out_hbm, ibuf, obuf, isem, osem):
    def in_copy(i):
        return pltpu.make_async_copy(
            in_hbm.at[pl.ds(i * _BLOCK, _BLOCK), :],
            ibuf.at[i],
            isem.at[i],
        )

    def out_copy(i):
        return pltpu.make_async_copy(
            obuf.at[i],
            out_hbm.at[pl.ds(i * _BLOCK, _BLOCK), :],
            osem.at[i],
        )

    for i in range(_NBLK):
        in_copy(i).start()
    for i in range(_NBLK):
        in_copy(i).wait()
        obuf[i] = ibuf[i][:, _COL_START:_COL_START + _COL_COUNT]
        out_copy(i).start()
    for i in range(_NBLK):
        out_copy(i).wait()


def kernel(inputs):
    n_rows, n_cols = inputs.shape
    return pl.pallas_call(
        _body,
        in_specs=[pl.BlockSpec(memory_space=pltpu.MemorySpace.HBM)],
        out_specs=pl.BlockSpec(memory_space=pltpu.MemorySpace.HBM),
        out_shape=jax.ShapeDtypeStruct((n_rows, _COL_COUNT), jnp.float32),
        scratch_shapes=[
            pltpu.VMEM((_NBLK, _BLOCK, 126), jnp.float32),
            pltpu.VMEM((_NBLK, _BLOCK, _COL_COUNT), jnp.float32),
            pltpu.SemaphoreType.DMA((_NBLK,)),
            pltpu.SemaphoreType.DMA((_NBLK,)),
        ],
    )(inputs)
